# Initial kernel scaffold; baseline (speedup 1.0000x reference)
import jax
import jax.numpy as jnp
from jax.experimental import pallas as pl


def kernel(in_feat, edge_index, Wp1, bp1, Wn1, Ws1, bs1, Wp2, bp2, Wn2, Ws2, bs2, Wl, bl):
    raise NotImplementedError("write your pallas kernel here")


# speedup vs baseline: 1.0315x; 1.0315x over previous
import functools

import jax
import jax.numpy as jnp
from jax import lax
from jax.experimental import pallas as pl
from jax.experimental.pallas import tpu as pltpu
from jax.experimental.pallas import tpu_sc as plsc

N = 10000
E = 320000
D = 128

NW = 32
R = 313
NP = NW * R
C = 4000
NCHUNK = E // C
NG = C // 16
FG = D // 16

_HIGH = lax.Precision.HIGHEST


def _stage_a_body(x_ref, wp_ref, bp_ref, ws_ref, bs_ref, hp_ref, self_ref):
    x = x_ref[...]
    hp = jnp.dot(x, wp_ref[...], preferred_element_type=jnp.float32,
                 precision=_HIGH) + bp_ref[...]
    hp_ref[...] = jnp.maximum(hp, 0.0)
    self_ref[...] = jnp.dot(x, ws_ref[...], preferred_element_type=jnp.float32,
                            precision=_HIGH) + bs_ref[...]


def _stage_b_body(self1_ref, neigh1_ref, wn1_ref, wp2_ref, bp2_ref,
                  ws2_ref, bs2_ref, hp2_ref, self2_ref):
    h = self1_ref[...] + jnp.dot(neigh1_ref[...], wn1_ref[...],
                                 preferred_element_type=jnp.float32,
                                 precision=_HIGH)
    h = jnp.where(h >= 0.0, h, 0.01 * h)
    hp2 = jnp.dot(h, wp2_ref[...], preferred_element_type=jnp.float32,
                  precision=_HIGH) + bp2_ref[...]
    hp2_ref[...] = jnp.maximum(hp2, 0.0)
    self2_ref[...] = jnp.dot(h, ws2_ref[...], preferred_element_type=jnp.float32,
                             precision=_HIGH) + bs2_ref[...]


def _stage_c_body(self2_ref, neigh2_ref, wn2_ref, wl_ref, bl_ref, out_ref):
    h = self2_ref[...] + jnp.dot(neigh2_ref[...], wn2_ref[...],
                                 preferred_element_type=jnp.float32,
                                 precision=_HIGH)
    h = jnp.where(h >= 0.0, h, 0.01 * h)
    logits = jnp.dot(h, wl_ref[...], preferred_element_type=jnp.float32,
                     precision=_HIGH) + bl_ref[...]
    out_ref[...] = jax.nn.sigmoid(logits)


_BN = 2000


def _row_spec(cols):
    return pl.BlockSpec((_BN, cols), lambda i: (i, 0))


def _full_spec(rows, cols):
    return pl.BlockSpec((rows, cols), lambda i: (0, 0))


def _stage_a(x, wp_t, bp, ws_t, bs):
    return pl.pallas_call(
        _stage_a_body,
        grid=(N // _BN,),
        in_specs=[_row_spec(D), _full_spec(D, D), _full_spec(1, D),
                  _full_spec(D, D), _full_spec(1, D)],
        out_specs=[_row_spec(D), _row_spec(D)],
        out_shape=[jax.ShapeDtypeStruct((N, D), jnp.float32),
                   jax.ShapeDtypeStruct((N, D), jnp.float32)],
    )(x, wp_t, bp, ws_t, bs)


def _stage_b(self1, neigh1, wn1_t, wp2_t, bp2, ws2_t, bs2):
    return pl.pallas_call(
        _stage_b_body,
        grid=(N // _BN,),
        in_specs=[_row_spec(D), _row_spec(D), _full_spec(D, D),
                  _full_spec(D, D), _full_spec(1, D),
                  _full_spec(D, D), _full_spec(1, D)],
        out_specs=[_row_spec(D), _row_spec(D)],
        out_shape=[jax.ShapeDtypeStruct((N, D), jnp.float32),
                   jax.ShapeDtypeStruct((N, D), jnp.float32)],
    )(self1, neigh1, wn1_t, wp2_t, bp2, ws2_t, bs2)


def _stage_c(self2, neigh2, wn2_t, wl_t, bl):
    nclass = wl_t.shape[1]
    return pl.pallas_call(
        _stage_c_body,
        grid=(N // _BN,),
        in_specs=[_row_spec(D), _row_spec(D), _full_spec(D, D),
                  _full_spec(D, nclass), _full_spec(1, nclass)],
        out_specs=_row_spec(nclass),
        out_shape=jax.ShapeDtypeStruct((N, nclass), jnp.float32),
    )(self2, neigh2, wn2_t, wl_t, bl)


def _segmax_body(hp_hbm, dst_hbm, src_hbm, out_hbm,
                 dbuf, sbuf, cd, cs, acc, rows, sem_e, sem_g):
    wid = lax.axis_index("s") * 2 + lax.axis_index("c")
    lo = wid * R

    def _zero(i, carry):
        acc[pl.ds(i * 16, 16)] = jnp.zeros((16,), jnp.float32)
        return carry
    lax.fori_loop(0, (R + 1) * D // 16, _zero, 0)

    def _chunk(c, carry):
        base_e = c * C
        pltpu.async_copy(dst_hbm.at[pl.ds(base_e, C)], dbuf, sem_e).wait()
        pltpu.async_copy(src_hbm.at[pl.ds(base_e, C)], sbuf, sem_e).wait()

        lane = lax.iota(jnp.int32, 16)

        def _scan(g, cursor):
            d = dbuf[pl.ds(g * 16, 16)]
            s = sbuf[pl.ds(g * 16, 16)]
            m = (d >= lo) & (d < lo + R)
            v = jnp.where(m, 1, 0)
            for st in (1, 2, 4, 8):
                sh = v[(lane - st) & 15]
                v = v + jnp.where(lane >= st, sh, 0)
            pos = jnp.zeros((16,), jnp.int32)
            for b in (8, 4, 2, 1):
                vc = v[pos + (b - 1)]
                pos = jnp.where(vc <= lane, pos + b, pos)
            cd[pl.ds(cursor, 16)] = d[pos]
            cs[pl.ds(cursor, 16)] = s[pos]
            return cursor + v[15]
        cursor = lax.fori_loop(0, NG, _scan, 0)

        cd[pl.ds(cursor, 16)] = jnp.full((16,), lo + R, jnp.int32)
        cs[pl.ds(cursor, 16)] = jnp.zeros((16,), jnp.int32)
        ngrp = (cursor + 15) // 16

        def _group(g, carry2):
            pltpu.async_copy(hp_hbm.at[cs.at[pl.ds(g * 16, 16)]], rows,
                             sem_g).wait()
            cdv = cd[pl.ds(g * 16, 16)]
            for j in range(16):
                row = cdv[j] - lo
                base = row * D
                for f in range(FG):
                    off = base + f * 16
                    cur = acc[pl.ds(off, 16)]
                    msg = rows[j, pl.ds(f * 16, 16)]
                    acc[pl.ds(off, 16)] = jnp.maximum(cur, msg)
            return carry2
        lax.fori_loop(0, ngrp, _group, 0)
        return carry
    lax.fori_loop(0, NCHUNK, _chunk, 0)

    pltpu.sync_copy(acc.at[pl.ds(0, R * D)], out_hbm.at[pl.ds(lo * D, R * D)])


@functools.partial(
    pl.kernel,
    out_type=jax.ShapeDtypeStruct((NP * D,), jnp.float32),
    mesh=plsc.VectorSubcoreMesh(core_axis_name="c", subcore_axis_name="s",
                                num_cores=2, num_subcores=16),
    scratch_types=[
        pltpu.VMEM((C,), jnp.int32),
        pltpu.VMEM((C,), jnp.int32),
        pltpu.VMEM((C + 16,), jnp.int32),
        pltpu.VMEM((C + 16,), jnp.int32),
        pltpu.VMEM(((R + 1) * D,), jnp.float32),
        pltpu.VMEM((16, D), jnp.float32),
        pltpu.SemaphoreType.DMA,
        pltpu.SemaphoreType.DMA,
    ],
)
def _segmax(hp, dst, src, out, *refs):
    _segmax_body(hp, dst, src, out, *refs)


def _segment_max(hp, src, dst):
    flat = _segmax(hp, dst, src)
    return flat.reshape(NP, D)[:N]


def kernel(in_feat, edge_index, Wp1, bp1, Wn1, Ws1, bs1,
           Wp2, bp2, Wn2, Ws2, bs2, Wl, bl):
    src = edge_index[0]
    dst = edge_index[1]

    hp1, self1 = _stage_a(in_feat, Wp1.T, bp1.reshape(1, D),
                          Ws1.T, bs1.reshape(1, D))
    neigh1 = _segment_max(hp1, src, dst)
    hp2, self2 = _stage_b(self1, neigh1, Wn1.T, Wp2.T, bp2.reshape(1, D),
                          Ws2.T, bs2.reshape(1, D))
    neigh2 = _segment_max(hp2, src, dst)
    out = _stage_c(self2, neigh2, Wn2.T, Wl.T, bl.reshape(1, -1))
    return out


# speedup vs baseline: 1.2284x; 1.1908x over previous
import functools

import jax
import jax.numpy as jnp
from jax import lax
from jax.experimental import pallas as pl
from jax.experimental.pallas import tpu as pltpu
from jax.experimental.pallas import tpu_sc as plsc

N = 10000
E = 320000
D = 128

NW = 32
R = 313
NP = NW * R
C = 4000
NCHUNK = E // C
NG = C // 16
FG = D // 16

_HIGH = lax.Precision.HIGHEST


def _stage_a_body(x_ref, wp_ref, bp_ref, ws_ref, bs_ref, hp_ref, self_ref):
    x = x_ref[...]
    hp = jnp.dot(x, wp_ref[...], preferred_element_type=jnp.float32,
                 precision=_HIGH) + bp_ref[...]
    hp_ref[...] = jnp.maximum(hp, 0.0)
    self_ref[...] = jnp.dot(x, ws_ref[...], preferred_element_type=jnp.float32,
                            precision=_HIGH) + bs_ref[...]


def _stage_b_body(self1_ref, neigh1_ref, wn1_ref, wp2_ref, bp2_ref,
                  ws2_ref, bs2_ref, hp2_ref, self2_ref):
    h = self1_ref[...] + jnp.dot(neigh1_ref[...], wn1_ref[...],
                                 preferred_element_type=jnp.float32,
                                 precision=_HIGH)
    h = jnp.where(h >= 0.0, h, 0.01 * h)
    hp2 = jnp.dot(h, wp2_ref[...], preferred_element_type=jnp.float32,
                  precision=_HIGH) + bp2_ref[...]
    hp2_ref[...] = jnp.maximum(hp2, 0.0)
    self2_ref[...] = jnp.dot(h, ws2_ref[...], preferred_element_type=jnp.float32,
                             precision=_HIGH) + bs2_ref[...]


def _stage_c_body(self2_ref, neigh2_ref, wn2_ref, wl_ref, bl_ref, out_ref):
    h = self2_ref[...] + jnp.dot(neigh2_ref[...], wn2_ref[...],
                                 preferred_element_type=jnp.float32,
                                 precision=_HIGH)
    h = jnp.where(h >= 0.0, h, 0.01 * h)
    logits = jnp.dot(h, wl_ref[...], preferred_element_type=jnp.float32,
                     precision=_HIGH) + bl_ref[...]
    out_ref[...] = jax.nn.sigmoid(logits)


_BN = 2000


def _row_spec(cols):
    return pl.BlockSpec((_BN, cols), lambda i: (i, 0))


def _full_spec(rows, cols):
    return pl.BlockSpec((rows, cols), lambda i: (0, 0))


def _stage_a(x, wp_t, bp, ws_t, bs):
    return pl.pallas_call(
        _stage_a_body,
        grid=(N // _BN,),
        in_specs=[_row_spec(D), _full_spec(D, D), _full_spec(1, D),
                  _full_spec(D, D), _full_spec(1, D)],
        out_specs=[_row_spec(D), _row_spec(D)],
        out_shape=[jax.ShapeDtypeStruct((N, D), jnp.float32),
                   jax.ShapeDtypeStruct((N, D), jnp.float32)],
    )(x, wp_t, bp, ws_t, bs)


def _stage_b(self1, neigh1, wn1_t, wp2_t, bp2, ws2_t, bs2):
    return pl.pallas_call(
        _stage_b_body,
        grid=(N // _BN,),
        in_specs=[_row_spec(D), _row_spec(D), _full_spec(D, D),
                  _full_spec(D, D), _full_spec(1, D),
                  _full_spec(D, D), _full_spec(1, D)],
        out_specs=[_row_spec(D), _row_spec(D)],
        out_shape=[jax.ShapeDtypeStruct((N, D), jnp.float32),
                   jax.ShapeDtypeStruct((N, D), jnp.float32)],
    )(self1, neigh1, wn1_t, wp2_t, bp2, ws2_t, bs2)


def _stage_c(self2, neigh2, wn2_t, wl_t, bl):
    nclass = wl_t.shape[1]
    return pl.pallas_call(
        _stage_c_body,
        grid=(N // _BN,),
        in_specs=[_row_spec(D), _row_spec(D), _full_spec(D, D),
                  _full_spec(D, nclass), _full_spec(1, nclass)],
        out_specs=_row_spec(nclass),
        out_shape=jax.ShapeDtypeStruct((N, nclass), jnp.float32),
    )(self2, neigh2, wn2_t, wl_t, bl)


def _segmax_body(hp_hbm, dst_hbm, src_hbm, out_hbm,
                 dbuf0, dbuf1, sbuf0, sbuf1, cd, cs, acc, rows0, rows1,
                 sem_e0, sem_e1, sem_g0, sem_g1):
    wid = lax.axis_index("s") * 2 + lax.axis_index("c")
    lo = wid * R
    lane = lax.iota(jnp.int32, 16)
    sem_e = (sem_e0, sem_e1)
    sem_g = (sem_g0, sem_g1)
    dbuf = (dbuf0, dbuf1)
    sbuf = (sbuf0, sbuf1)
    rows = (rows0, rows1)

    def _zero(i, carry):
        for u in range(4):
            acc[pl.ds(i * 64 + u * 16, 16)] = jnp.zeros((16,), jnp.float32)
        return carry
    lax.fori_loop(0, (R + 1) * D // 64, _zero, 0)

    def _start_chunk(c, b):
        base_e = c * C
        pltpu.async_copy(dst_hbm.at[pl.ds(base_e, C)], dbuf[b], sem_e[b])
        pltpu.async_copy(src_hbm.at[pl.ds(base_e, C)], sbuf[b], sem_e[b])

    def _wait_chunk(b):
        pltpu.make_async_copy(dst_hbm.at[pl.ds(0, C)], dbuf[b],
                              sem_e[b]).wait()
        pltpu.make_async_copy(src_hbm.at[pl.ds(0, C)], sbuf[b],
                              sem_e[b]).wait()

    def _scan_chunk(b):
        def _scan(g, cursor):
            d = dbuf[b][pl.ds(g * 16, 16)]
            s = sbuf[b][pl.ds(g * 16, 16)]
            m = (d >= lo) & (d < lo + R)
            v = jnp.where(m, 1, 0)
            for st in (1, 2, 4, 8):
                sh = v[(lane - st) & 15]
                v = v + jnp.where(lane >= st, sh, 0)
            cnt = v[15]

            def _compact(cur):
                pos = jnp.zeros((16,), jnp.int32)
                for bb in (8, 4, 2, 1):
                    vc = v[pos + (bb - 1)]
                    pos = jnp.where(vc <= lane, pos + bb, pos)
                cd[pl.ds(cur, 16)] = d[pos]
                cs[pl.ds(cur, 16)] = s[pos]
                return cur + cnt
            return lax.cond(cnt > 0, _compact, lambda cur: cur, cursor)
        return lax.fori_loop(0, NG, _scan, 0)

    def _gstart(g, b):
        pltpu.async_copy(hp_hbm.at[cs.at[pl.ds(g * 16, 16)]], rows[b],
                         sem_g[b])

    def _gwait(b):
        pltpu.make_async_copy(hp_hbm.at[cs.at[pl.ds(0, 16)]], rows[b],
                              sem_g[b]).wait()

    def _proc(g, b):
        cdv = cd[pl.ds(g * 16, 16)]
        for j in range(16):
            base = (cdv[j] - lo) * D
            for f in range(FG):
                off = base + f * 16
                acc[pl.ds(off, 16)] = jnp.maximum(
                    acc[pl.ds(off, 16)], rows[b][j, pl.ds(f * 16, 16)])

    def _process_chunk(cursor):
        cd[pl.ds(cursor, 16)] = jnp.full((16,), lo + R, jnp.int32)
        cs[pl.ds(cursor, 16)] = jnp.zeros((16,), jnp.int32)
        ngrp = (cursor + 15) // 16
        _gstart(0, 0)

        def _pair(i, carry2):
            g = i * 2

            @pl.when(g + 1 < ngrp)
            def _():
                _gstart(g + 1, 1)
            _gwait(0)
            _proc(g, 0)

            @pl.when(g + 1 < ngrp)
            def _():
                @pl.when(g + 2 < ngrp)
                def _():
                    _gstart(g + 2, 0)
                _gwait(1)
                _proc(g + 1, 1)
            return carry2
        lax.fori_loop(0, (ngrp + 1) // 2, _pair, 0)

    def _handle_chunk(b):
        cursor = _scan_chunk(b)

        @pl.when(cursor > 0)
        def _():
            _process_chunk(cursor)

    _start_chunk(0, 0)

    def _cpair(i, carry):
        c = i * 2

        @pl.when(c + 1 < NCHUNK)
        def _():
            _start_chunk(c + 1, 1)
        _wait_chunk(0)
        _handle_chunk(0)

        @pl.when(c + 2 < NCHUNK)
        def _():
            _start_chunk(c + 2, 0)
        _wait_chunk(1)
        _handle_chunk(1)
        return carry
    lax.fori_loop(0, NCHUNK // 2, _cpair, 0)

    pltpu.sync_copy(acc.at[pl.ds(0, R * D)], out_hbm.at[pl.ds(lo * D, R * D)])


@functools.partial(
    pl.kernel,
    out_type=jax.ShapeDtypeStruct((NP * D,), jnp.float32),
    mesh=plsc.VectorSubcoreMesh(core_axis_name="c", subcore_axis_name="s",
                                num_cores=2, num_subcores=16),
    scratch_types=[
        pltpu.VMEM((C,), jnp.int32),
        pltpu.VMEM((C,), jnp.int32),
        pltpu.VMEM((C,), jnp.int32),
        pltpu.VMEM((C,), jnp.int32),
        pltpu.VMEM((C + 16,), jnp.int32),
        pltpu.VMEM((C + 16,), jnp.int32),
        pltpu.VMEM(((R + 1) * D,), jnp.float32),
        pltpu.VMEM((16, D), jnp.float32),
        pltpu.VMEM((16, D), jnp.float32),
        pltpu.SemaphoreType.DMA,
        pltpu.SemaphoreType.DMA,
        pltpu.SemaphoreType.DMA,
        pltpu.SemaphoreType.DMA,
    ],
)
def _segmax(hp, dst, src, out, *refs):
    _segmax_body(hp, dst, src, out, *refs)


def _segment_max(hp, src, dst):
    flat = _segmax(hp, dst, src)
    return flat.reshape(NP, D)[:N]


def kernel(in_feat, edge_index, Wp1, bp1, Wn1, Ws1, bs1,
           Wp2, bp2, Wn2, Ws2, bs2, Wl, bl):
    src = edge_index[0]
    dst = edge_index[1]

    hp1, self1 = _stage_a(in_feat, Wp1.T, bp1.reshape(1, D),
                          Ws1.T, bs1.reshape(1, D))
    neigh1 = _segment_max(hp1, src, dst)
    hp2, self2 = _stage_b(self1, neigh1, Wn1.T, Wp2.T, bp2.reshape(1, D),
                          Ws2.T, bs2.reshape(1, D))
    neigh2 = _segment_max(hp2, src, dst)
    out = _stage_c(self2, neigh2, Wn2.T, Wl.T, bl.reshape(1, -1))
    return out
